# baseline (speedup 1.0000x reference)
import functools

import jax
import jax.numpy as jnp
from jax import lax
from jax.experimental import pallas as pl
from jax.experimental.pallas import tpu as pltpu
from jax.experimental.pallas import tpu_sc as plsc

N_COINS = 100000
EMBED_DIM = 64
BATCH = 16384
HIST = 50

NC, NS = 2, 16
NW = NC * NS
BPW = BATCH // NW
LT_PER_W = BPW // 128
BLOCKS = HIST * LT_PER_W
LANES = 16


def _transpose_block(rows, tile):
    iota = lax.iota(jnp.int32, LANES)
    zero = iota * 0
    for d in range(EMBED_DIM):
        d_vec = zero + d
        for g in range(8):
            zl = iota + (g * LANES)
            v = plsc.load_gather(rows, [zl, d_vec])
            tile[d // 8, d % 8, pl.ds(g * LANES, LANES)] = v


def _gather_kernel(table_hbm, idx_hbm, out_hbm,
                   idx_v, ilist0, ilist1, rows0, rows1, tile0, tile1,
                   gsem0, gsem1, wsem0, wsem1):
    wid = lax.axis_index("s") * NC + lax.axis_index("c")
    ilist = (ilist0, ilist1)
    rows = (rows0, rows1)
    tile = (tile0, tile1)
    gsem = (gsem0, gsem1)
    wsem = (wsem0, wsem1)

    pltpu.sync_copy(idx_hbm.at[pl.ds(wid * BPW, BPW)], idx_v)

    iota = lax.iota(jnp.int32, LANES)

    def stage_a(c, s):
        h = c // LT_PER_W
        ltl = c % LT_PER_W
        h_vec = iota * 0 + h
        for g in range(8):
            b_vec = iota + (ltl * 128 + g * LANES)
            v = plsc.load_gather(idx_v, [b_vec, h_vec])
            ilist[s][pl.ds(g * LANES, LANES)] = v
        pltpu.async_copy(table_hbm.at[ilist[s]], rows[s], gsem[s])

    def stage_b(c, s):
        h = c // LT_PER_W
        ltl = c % LT_PER_W
        pltpu.make_async_copy(table_hbm.at[pl.ds(0, 128)], rows[s],
                              gsem[s]).wait()
        _transpose_block(rows[s], tile[s])
        pltpu.async_copy(tile[s], out_hbm.at[h, :, wid * LT_PER_W + ltl],
                         wsem[s])

    def stage_c(s):
        pltpu.make_async_copy(tile[s], out_hbm.at[0, :, 0], wsem[s]).wait()

    stage_a(0, 0)
    stage_a(1, 1)
    stage_b(0, 0)

    def body(j, carry):
        k1 = 2 * j + 1
        stage_a(k1 + 1, 0)
        stage_b(k1, 1)
        stage_c(0)
        stage_a(k1 + 2, 1)
        stage_b(k1 + 1, 0)
        stage_c(1)
        return carry

    lax.fori_loop(0, (BLOCKS - 2) // 2, body, 0)

    stage_b(BLOCKS - 1, 1)
    stage_c(0)
    stage_c(1)


@functools.cache
def _build():
    return pl.kernel(
        _gather_kernel,
        out_type=jax.ShapeDtypeStruct((HIST, 8, BATCH // 128, 8, 128),
                                      jnp.float32),
        mesh=plsc.VectorSubcoreMesh(
            core_axis_name="c", subcore_axis_name="s",
            num_cores=NC, num_subcores=NS,
        ),
        scratch_types=[
            pltpu.VMEM((BPW, HIST), jnp.int32),
            pltpu.VMEM((128,), jnp.int32),
            pltpu.VMEM((128,), jnp.int32),
            pltpu.VMEM((128, EMBED_DIM), jnp.float32),
            pltpu.VMEM((128, EMBED_DIM), jnp.float32),
            pltpu.VMEM((8, 8, 128), jnp.float32),
            pltpu.VMEM((8, 8, 128), jnp.float32),
            pltpu.SemaphoreType.DMA,
            pltpu.SemaphoreType.DMA,
            pltpu.SemaphoreType.DMA,
            pltpu.SemaphoreType.DMA,
        ],
        compiler_params=pltpu.CompilerParams(use_tc_tiling_on_sc=False,
                                             needs_layout_passes=False),
    )


def kernel(coin_id, table):
    out5 = _build()(table, coin_id.astype(jnp.int32))
    return out5.transpose(2, 4, 0, 1, 3).reshape(BATCH, HIST, EMBED_DIM)


# speedup vs baseline: 3.3655x; 3.3655x over previous
import functools

import jax
import jax.numpy as jnp
from jax import lax
from jax.experimental import pallas as pl
from jax.experimental.pallas import tpu as pltpu
from jax.experimental.pallas import tpu_sc as plsc

N_COINS = 100000
EMBED_DIM = 64
BATCH = 16384
HIST = 50
HIST_P = 56
DIM_P = 128

NC, NS = 2, 16
NW = NC * NS
BATCH_PER_W = BATCH // NW
NB = 8
CHUNKS_PER_W = BATCH_PER_W // NB
NBUF = 2


def _gather_kernel(table_hbm, idx_hbm, out_hbm,
                   idx_v, rows0, rows1, gsem0, gsem1, wsem0, wsem1):
    wid = lax.axis_index("s") * NC + lax.axis_index("c")
    batch0 = wid * BATCH_PER_W
    rows = (rows0, rows1)
    gsem = (gsem0, gsem1)
    wsem = (wsem0, wsem1)

    pltpu.sync_copy(idx_hbm.at[pl.ds(batch0, BATCH_PER_W)], idx_v)

    def fire_gather(c, s):
        for b in range(NB):
            pltpu.async_copy(
                table_hbm.at[idx_v.at[c * NB + b]],
                rows[s].at[b],
                gsem[s])

    def drain_gather(s):
        pltpu.make_async_copy(
            out_hbm.at[pl.ds(0, NB), pl.ds(0, HIST), pl.ds(0, EMBED_DIM)],
            rows[s],
            gsem[s]).wait()

    def fire_write(c, s):
        pltpu.async_copy(
            rows[s],
            out_hbm.at[pl.ds(batch0 + c * NB, NB), pl.ds(0, HIST),
                       pl.ds(0, EMBED_DIM)],
            wsem[s])

    def drain_write(s):
        pltpu.make_async_copy(
            rows[s],
            out_hbm.at[pl.ds(batch0, NB), pl.ds(0, HIST),
                       pl.ds(0, EMBED_DIM)],
            wsem[s]).wait()

    for s in range(NBUF):
        fire_gather(s, s)

    def body(i, carry):
        g = i * NBUF
        for s in range(NBUF):
            c = g + s
            drain_gather(s)
            fire_write(c, s)
            drain_write(s)
            fire_gather(c + NBUF, s)
        return carry

    lax.fori_loop(0, (CHUNKS_PER_W - NBUF) // NBUF, body, 0)

    for s in range(NBUF):
        drain_gather(s)
        fire_write(CHUNKS_PER_W - NBUF + s, s)
        drain_write(s)


@functools.cache
def _build():
    return pl.kernel(
        _gather_kernel,
        out_type=jax.ShapeDtypeStruct((BATCH, HIST_P, DIM_P), jnp.float32),
        mesh=plsc.VectorSubcoreMesh(
            core_axis_name="c", subcore_axis_name="s",
            num_cores=NC, num_subcores=NS,
        ),
        scratch_types=[
            pltpu.VMEM((BATCH_PER_W, HIST), jnp.int32),
            pltpu.VMEM((NB, HIST, EMBED_DIM), jnp.float32),
            pltpu.VMEM((NB, HIST, EMBED_DIM), jnp.float32),
            pltpu.SemaphoreType.DMA,
            pltpu.SemaphoreType.DMA,
            pltpu.SemaphoreType.DMA,
            pltpu.SemaphoreType.DMA,
        ],
        compiler_params=pltpu.CompilerParams(use_tc_tiling_on_sc=False),
    )


def kernel(coin_id, table):
    out_p = _build()(table, coin_id.astype(jnp.int32))
    return out_p[:, :HIST, :EMBED_DIM]
